# Initial kernel scaffold; baseline (speedup 1.0000x reference)
import functools

import jax
import jax.numpy as jnp
from jax import lax
from jax.experimental import pallas as pl
from jax.experimental.pallas import tpu as pltpu
from jax.experimental.pallas import tpu_sc as plsc

NN = 50000
CD = 40
HID = 64
HEADS = 4
CH = 16
NP = 50016
NT = 16
EB = 128
BN = 1000


def _sc_body(etype, ept, nblk, idx_hbm, attr_hbm, xl_hbm, xr_hbm, par_hbm,
             zn_hbm, zd_hbm, num_out, den_out,
             idxv, gsrcv, gdstv, dvv, attrv, xlb, xrb, nums, dens,
             wesv, lgv, exv, accn, accd, sem_a, sem_b):
    c = lax.axis_index("c")
    s = lax.axis_index("s")
    rpt = NP // NT
    r0 = s * rpt
    pltpu.sync_copy(par_hbm.at[c], wesv)
    pltpu.sync_copy(zn_hbm.at[pl.ds(r0, rpt)], accn.at[pl.ds(r0, rpt)])
    pltpu.sync_copy(zd_hbm.at[pl.ds(r0, rpt)], accd.at[pl.ds(r0, rpt)])
    plsc.subcore_barrier()
    e0 = s * ept
    iot = lax.iota(jnp.int32, (16,))
    lane15 = iot == 15
    zeros16 = jnp.zeros((16,), jnp.int32)
    ones16 = jnp.ones((16,), jnp.int32)

    def blk(b, carry):
        base = e0 + b * EB
        d1 = pltpu.async_copy(idx_hbm.at[pl.ds(base * 4, EB * 4)], idxv, sem_a)
        d2 = pltpu.async_copy(attr_hbm.at[pl.ds(base, EB)], attrv, sem_a)
        d1.wait()
        d2.wait()
        for g in range(EB // 16):
            i4 = (g * 16 + iot) * 4
            sv = plsc.load_gather(idxv, [i4])
            dv = plsc.load_gather(idxv, [i4 + 1])
            tv = plsc.load_gather(idxv, [i4 + 2])
            gsrcv[pl.ds(g * 16, 16)] = sv * 2 + c
            gdstv[pl.ds(g * 16, 16)] = dv * 2 + c
            dvv[pl.ds(g * 16, 16)] = jnp.where(tv == etype, dv, NN)
        ga = pltpu.async_copy(xl_hbm.at[gsrcv], xlb, sem_b)
        gb = pltpu.async_copy(xr_hbm.at[gdstv], xrb, sem_b)
        ga.wait()
        gb.wait()

        w0h = (wesv[pl.ds(0, 16)], wesv[pl.ds(16, 16)])
        w1h = (wesv[pl.ds(32, 16)], wesv[pl.ds(48, 16)])
        ath = (wesv[pl.ds(64, 16)], wesv[pl.ds(80, 16)])

        def grp(g, carry2):
            for j in range(16):
                e = g * 16 + j
                a0 = attrv[e, 0]
                a1 = attrv[e, 1]
                for h in range(2):
                    m = (xlb[e, pl.ds(16 * h, 16)] + xrb[e, pl.ds(16 * h, 16)]
                         + a0 * w0h[h] + a1 * w1h[h])
                    m = jnp.where(m >= 0.0, m, 0.2 * m)
                    cum = plsc.cumsum(m * ath[h])
                    plsc.store_scatter(
                        lgv, [jnp.full((16,), 16 * h + j, jnp.int32)], cum,
                        mask=lane15)
            ex0 = jnp.exp(lgv[pl.ds(0, 16)])
            ex1 = jnp.exp(lgv[pl.ds(16, 16)])
            exv[pl.ds(0, 16)] = ex0
            exv[pl.ds(16, 16)] = ex1
            row = g * 16 + iot
            plsc.store_scatter(dens, [row, zeros16], ex0)
            plsc.store_scatter(dens, [row, ones16], ex1)
            for j in range(16):
                e = g * 16 + j
                for h in range(2):
                    nums[e, pl.ds(16 * h, 16)] = (
                        exv[16 * h + j] * xlb[e, pl.ds(16 * h, 16)])
            return 0

        lax.fori_loop(0, EB // 16, grp, 0)
        pltpu.sync_copy(nums, accn.at[dvv], add=True)
        pltpu.sync_copy(dens, accd.at[dvv], add=True)
        return 0

    lax.fori_loop(0, nblk, blk, 0)
    plsc.subcore_barrier()
    pltpu.sync_copy(accn.at[pl.ds(r0, rpt)], num_out.at[pl.ds(r0, rpt), c])
    pltpu.sync_copy(accd.at[pl.ds(r0, rpt)], den_out.at[pl.ds(r0, rpt), c])


@functools.cache
def _make_sc(etype, epad):
    ept = epad // NT
    nblk = ept // EB
    mesh = plsc.VectorSubcoreMesh(core_axis_name="c", subcore_axis_name="s")
    return pl.kernel(
        functools.partial(_sc_body, etype, ept, nblk),
        out_type=[jax.ShapeDtypeStruct((NP, 2, 32), jnp.float32),
                  jax.ShapeDtypeStruct((NP, 2, 2), jnp.float32)],
        mesh=mesh,
        scratch_types=[
            pltpu.VMEM((EB * 4,), jnp.int32),
            pltpu.VMEM((EB,), jnp.int32),
            pltpu.VMEM((EB,), jnp.int32),
            pltpu.VMEM((EB,), jnp.int32),
            pltpu.VMEM((EB, 2), jnp.float32),
            pltpu.VMEM((EB, 32), jnp.float32),
            pltpu.VMEM((EB, 32), jnp.float32),
            pltpu.VMEM((EB, 32), jnp.float32),
            pltpu.VMEM((EB, 2), jnp.float32),
            pltpu.VMEM((96,), jnp.float32),
            pltpu.VMEM((32,), jnp.float32),
            pltpu.VMEM((32,), jnp.float32),
            pltpu.VMEM_SHARED((NP, 32), jnp.float32),
            pltpu.VMEM_SHARED((NP, 2), jnp.float32),
            pltpu.SemaphoreType.DMA,
            pltpu.SemaphoreType.DMA,
        ],
        name="gat_edge_pass_t%d" % etype,
    )


def _mean_body(a0_ref, a1_ref, et_ref, out_ref):
    i = pl.program_id(0)

    @pl.when(i == 0)
    def _():
        out_ref[...] = jnp.zeros_like(out_ref)

    a0 = a0_ref[0]
    a1 = a1_ref[0]
    et = et_ref[0]
    m0 = (et == 0).astype(jnp.float32)
    m1 = (et == 1).astype(jnp.float32)
    lane = lax.broadcasted_iota(jnp.int32, (1, 128), 1)
    z = jnp.zeros((1, 128), jnp.float32)
    vals = (jnp.where(lane == 0, jnp.sum(a0 * m0), z)
            + jnp.where(lane == 1, jnp.sum(a1 * m0), z)
            + jnp.where(lane == 2, jnp.sum(m0), z)
            + jnp.where(lane == 3, jnp.sum(a0 * m1), z)
            + jnp.where(lane == 4, jnp.sum(a1 * m1), z)
            + jnp.where(lane == 5, jnp.sum(m1), z))
    out_ref[...] += vals


def _edge_means(edge_type, edge_attr):
    e = edge_type.shape[0]
    be = 8000
    g = e // be
    a0 = edge_attr[:, 0].reshape(g, 1, be)
    a1 = edge_attr[:, 1].reshape(g, 1, be)
    et = edge_type.reshape(g, 1, be)
    sums = pl.pallas_call(
        _mean_body,
        grid=(g,),
        in_specs=[pl.BlockSpec((1, 1, be), lambda i: (i, 0, 0))] * 3,
        out_specs=pl.BlockSpec((1, 128), lambda i: (0, 0)),
        out_shape=jax.ShapeDtypeStruct((1, 128), jnp.float32),
    )(a0, a1, et)
    ma0 = sums[:, 0:2] / sums[0, 2]
    ma1 = sums[:, 3:5] / sums[0, 5]
    return ma0, ma1


def _prep_body(x_ref, fi_ref, ci_ref, fe_ref, ce_ref, ew_ref, eb_ref, pw_ref,
               w0l_ref, w0r_ref, w1l_ref, w1r_ref,
               xp_ref, xl0_ref, xr0_ref, xl1_ref, xr1_ref):
    b = x_ref.shape[0]
    fi = fi_ref[0]
    ci = ci_ref[0]
    ohf = (lax.broadcasted_iota(jnp.int32, (128, b), 0) == fi
           ).astype(jnp.float32)
    ohc = (lax.broadcasted_iota(jnp.int32, (32, b), 0) == ci
           ).astype(jnp.float32)
    dn = (((0,), (0,)), ((), ()))
    fe = lax.dot_general(ohf, fe_ref[...], dn,
                         preferred_element_type=jnp.float32)
    ce = lax.dot_general(ohc, ce_ref[...], dn,
                         preferred_element_type=jnp.float32)
    ew = ew_ref[...]
    xf = (x_ref[...] @ ew[:CD] + fe @ ew[CD:CD + 8] + ce @ ew[CD + 8:CD + 16]
          + eb_ref[...])
    xp_ref[...] = xf @ pw_ref[...]
    xl0_ref[...] = xf @ w0l_ref[...]
    xr0_ref[...] = xf @ w0r_ref[...]
    xl1_ref[...] = xf @ w1l_ref[...]
    xr1_ref[...] = xf @ w1r_ref[...]


def _prep(x, flag_idx, class_idx, p):
    nb = NN // BN
    cep = jnp.zeros((32, 8), jnp.float32).at[:17].set(p['class_emb'])
    fi3 = flag_idx.reshape(nb, 1, BN)
    ci3 = class_idx.reshape(nb, 1, BN)
    full = lambda shp: pl.BlockSpec(shp, lambda i: tuple(0 for _ in shp))
    row = lambda d: pl.BlockSpec((BN, d), lambda i: (i, 0))
    outs = pl.pallas_call(
        _prep_body,
        grid=(nb,),
        in_specs=[row(CD),
                  pl.BlockSpec((1, 1, BN), lambda i: (i, 0, 0)),
                  pl.BlockSpec((1, 1, BN), lambda i: (i, 0, 0)),
                  full((128, 8)), full((32, 8)), full((56, CD)),
                  full((1, CD)), full((CD, HID)),
                  full((CD, HID)), full((CD, HID)),
                  full((CD, HID)), full((CD, HID))],
        out_specs=[row(HID)] * 5,
        out_shape=[jax.ShapeDtypeStruct((NN, HID), jnp.float32)] * 5,
    )(x, fi3, ci3, p['flag_emb'], cep, p['emb_W'],
      p['emb_b'].reshape(1, CD), p['input_proj_W'],
      p['conv1_0']['Wl'], p['conv1_0']['Wr'],
      p['conv1_1']['Wl'], p['conv1_1']['Wr'])
    return outs


def _post_body(mid, num_ref, den_ref, xl_ref, xr_ref, res_ref,
               ma_ref, we_ref, att_ref, bias_ref, g_ref, b_ref, *rest):
    xl = xl_ref[...]
    xr = xr_ref[...]
    es = lax.dot_general(ma_ref[...], we_ref[...], (((1,), (0,)), ((), ())),
                         preferred_element_type=jnp.float32)
    ms = xl + xr + es
    ms = jnp.where(ms >= 0.0, ms, 0.2 * ms)
    sel = (lax.broadcasted_iota(jnp.int32, (HID, HEADS), 0) // CH
           == lax.broadcasted_iota(jnp.int32, (HID, HEADS), 1)
           ).astype(jnp.float32)
    dn_c1 = (((1,), (0,)), ((), ()))
    dn_c11 = (((1,), (1,)), ((), ()))
    ls = lax.dot_general(ms * att_ref[...], sel, dn_c1,
                         preferred_element_type=jnp.float32)
    exs = jnp.exp(ls)
    den4 = den_ref[...] + exs
    dexp = lax.dot_general(exs, sel, dn_c11,
                           preferred_element_type=jnp.float32)
    denx = lax.dot_general(den4, sel, dn_c11,
                           preferred_element_type=jnp.float32)
    out = (num_ref[...] + dexp * xl) / denx + bias_ref[...]
    hv = jnp.where(out > 0.0, out, jnp.exp(out) - 1.0)
    t = hv + res_ref[...]
    mu = jnp.mean(t, axis=-1, keepdims=True)
    var = jnp.mean((t - mu) ** 2, axis=-1, keepdims=True)
    hn = (t - mu) / jnp.sqrt(var + 1e-5) * g_ref[...] + b_ref[...]
    if mid:
        w2l_ref, w2r_ref, h_ref, xl2_ref, xr2_ref = rest
        h_ref[...] = hn
        xl2_ref[...] = hn @ w2l_ref[...]
        xr2_ref[...] = hn @ w2r_ref[...]
    else:
        rest[0][...] = hn


def _post(mid, num64, den4, xlt, xrt, res, ma, cp, gamma, beta, w2=None):
    nb = NN // BN
    full = lambda shp: pl.BlockSpec(shp, lambda i: tuple(0 for _ in shp))
    row = lambda d: pl.BlockSpec((BN, d), lambda i: (i, 0))
    n_out = 3 if mid else 1
    ins = [num64, den4, xlt, xrt, res, ma, cp['We'],
           cp['att'].reshape(1, HID), cp['bias'].reshape(1, HID),
           gamma.reshape(1, HID), beta.reshape(1, HID)]
    in_specs = [row(HID), row(HEADS), row(HID), row(HID), row(HID),
                full((1, 2)), full((2, HID)), full((1, HID)), full((1, HID)),
                full((1, HID)), full((1, HID))]
    if mid:
        ins += [w2['Wl'], w2['Wr']]
        in_specs += [full((HID, HID)), full((HID, HID))]
    outs = pl.pallas_call(
        functools.partial(_post_body, mid),
        grid=(nb,),
        in_specs=in_specs,
        out_specs=[row(HID)] * n_out,
        out_shape=[jax.ShapeDtypeStruct((NN, HID), jnp.float32)] * n_out,
    )(*ins)
    return outs if mid else outs[0]


def _final_body(h0_ref, h1_ref, tw_ref, pw_ref, pb_ref,
                wr_ref, wz_ref, wn_ref, br_ref, bz_ref, bn_ref,
                bhr_ref, bhz_ref, bhn_ref, gg_ref, gb_ref,
                hw_ref, hb_ref, o_ref):
    o64 = h0_ref[...] * tw_ref[0, 0] + h1_ref[...] * tw_ref[0, 1]
    s = o64 @ pw_ref[...] + pb_ref[...]
    r = jax.nn.sigmoid(s @ wr_ref[...] + br_ref[...] + bhr_ref[...])
    z = jax.nn.sigmoid(s @ wz_ref[...] + bz_ref[...] + bhz_ref[...])
    nc = jnp.tanh(s @ wn_ref[...] + bn_ref[...] + r * bhn_ref[...])
    hn = (1.0 - z) * nc
    mu = jnp.mean(hn, axis=-1, keepdims=True)
    var = jnp.mean((hn - mu) ** 2, axis=-1, keepdims=True)
    hn = (hn - mu) / jnp.sqrt(var + 1e-5) * gg_ref[...] + gb_ref[...]
    o_ref[...] = hn @ hw_ref[...] + hb_ref[...]


def _final(h20, h21, p):
    nb = NN // BN
    tw = jax.nn.softmax(p['type_weights']).reshape(1, 2)
    full = lambda shp: pl.BlockSpec(shp, lambda i: tuple(0 for _ in shp))
    row = lambda d: pl.BlockSpec((BN, d), lambda i: (i, 0))
    wih = p['gru_Wih']
    bih = p['gru_bih'].reshape(1, 3 * HID)
    bhh = p['gru_bhh'].reshape(1, 3 * HID)
    return pl.pallas_call(
        _final_body,
        grid=(nb,),
        in_specs=[row(HID), row(HID),
                  pl.BlockSpec(memory_space=pltpu.SMEM),
                  full((HID, HID)), full((1, HID)),
                  full((HID, HID)), full((HID, HID)), full((HID, HID)),
                  full((1, HID)), full((1, HID)), full((1, HID)),
                  full((1, HID)), full((1, HID)), full((1, HID)),
                  full((1, HID)), full((1, HID)),
                  full((HID, 4)), full((1, 4))],
        out_specs=row(4),
        out_shape=jax.ShapeDtypeStruct((NN, 4), jnp.float32),
    )(h20, h21, tw, p['proj_W'], p['proj_b'].reshape(1, HID),
      wih[:, :HID], wih[:, HID:2 * HID], wih[:, 2 * HID:],
      bih[:, :HID], bih[:, HID:2 * HID], bih[:, 2 * HID:],
      bhh[:, :HID], bhh[:, HID:2 * HID], bhh[:, 2 * HID:],
      p['gru_norm_g'].reshape(1, HID), p['gru_norm_b'].reshape(1, HID),
      p['head_W'], p['head_b'].reshape(1, 4))


def _pack_par(cp):
    we = cp['We']
    attf = cp['att'].reshape(HID)
    return jnp.stack([
        jnp.concatenate([we[0, :32], we[1, :32], attf[:32]]),
        jnp.concatenate([we[0, 32:], we[1, 32:], attf[32:]]),
    ])


def kernel(x, edge_index, edge_type, edge_attr, flag_idx, class_idx, params):
    e = edge_index.shape[1]
    quant = NT * EB
    epad = ((e + quant - 1) // quant) * quant
    pad = epad - e
    idx_pack = jnp.stack(
        [edge_index[0], edge_index[1], edge_type,
         jnp.zeros_like(edge_type)], axis=1)
    if pad:
        padrow = jnp.broadcast_to(
            jnp.array([0, 0, -1, 0], jnp.int32), (pad, 4))
        idx_pack = jnp.concatenate([idx_pack, padrow])
        attr_p = jnp.concatenate(
            [edge_attr, jnp.zeros((pad, 2), jnp.float32)])
    else:
        attr_p = edge_attr
    idx_flat = idx_pack.reshape(-1)
    zn = jnp.zeros((NP, 32), jnp.float32)
    zd = jnp.zeros((NP, 2), jnp.float32)

    ma = _edge_means(edge_type, edge_attr)
    xp, xl0, xr0, xl1, xr1 = _prep(x, flag_idx, class_idx, params)

    h2n = []
    for et, xlt, xrt in ((0, xl0, xr0), (1, xl1, xr1)):
        sck = _make_sc(et, epad)
        res = xp
        for li in range(1, 3):
            cp = params['conv%d_%d' % (li, et)]
            par = _pack_par(cp)
            num, den = sck(idx_flat, attr_p,
                           xlt.reshape(2 * NN, 32), xrt.reshape(2 * NN, 32),
                           par, zn, zd)
            num64 = num.reshape(NP, HID)
            den4 = den.reshape(NP, HEADS)
            g = params['norm%d_%d_g' % (li, et)]
            b = params['norm%d_%d_b' % (li, et)]
            if li == 1:
                h, xlt, xrt = _post(True, num64, den4, xlt, xrt, res,
                                    ma[et], cp, g, b,
                                    params['conv2_%d' % et])
                res = h
            else:
                h2n.append(_post(False, num64, den4, xlt, xrt, res,
                                 ma[et], cp, g, b))
    return _final(h2n[0], h2n[1], params)


# speedup vs baseline: 38.9499x; 38.9499x over previous
import functools

import jax
import jax.numpy as jnp
from jax import lax
from jax.experimental import pallas as pl
from jax.experimental.pallas import tpu as pltpu
from jax.experimental.pallas import tpu_sc as plsc

NN = 50000
CD = 40
HID = 64
HEADS = 4
CH = 16
NP = 51200
NT = 16
EB = 128
BN = 1000


def _sc_body(etype, ept, nblk, src_hbm, dst_hbm, et_hbm, attr_hbm,
             xl_hbm, xr_hbm, par_hbm,
             num_out, den_out,
             srcv, dstv, etv, gsrcv, gdstv, dvv, attrv, xlb, xrb, nums, dens,
             wesv, lgv, accn, accd, sem_a, sem_b):
    c = lax.axis_index("c")
    s = lax.axis_index("s")
    rpt = NP // NT
    r0 = s * rpt
    pltpu.sync_copy(par_hbm.at[c], wesv)
    iot = lax.iota(jnp.int32, 16)
    lane15 = iot == 15
    zeros16 = jnp.zeros((16,), jnp.int32)
    ones16 = jnp.ones((16,), jnp.int32)
    zf = jnp.zeros((16,), jnp.float32)

    def z_n(e, _):
        nums[e, pl.ds(0, 16)] = zf
        nums[e, pl.ds(16, 16)] = zf
        return 0

    lax.fori_loop(0, EB, z_n, 0)

    def z_d(k, _):
        plsc.store_scatter(dens, [k * 8 + iot // 2, iot % 2], zf)
        return 0

    lax.fori_loop(0, EB // 8, z_d, 0)

    def fill_dvv(o):
        for g in range(EB // 16):
            dvv[pl.ds(g * 16, 16)] = o + g * 16 + iot

    def init_blk(k, _):
        fill_dvv(r0 + k * EB)
        pltpu.sync_copy(nums, accn.at[dvv])
        pltpu.sync_copy(dens, accd.at[dvv])
        return 0

    lax.fori_loop(0, rpt // EB, init_blk, 0)
    plsc.subcore_barrier()
    e0 = s * ept

    def blk(b, carry):
        base = e0 + b * EB
        d0 = pltpu.async_copy(src_hbm.at[pl.ds(base, EB)], srcv, sem_a)
        d1 = pltpu.async_copy(dst_hbm.at[pl.ds(base, EB)], dstv, sem_a)
        d2 = pltpu.async_copy(et_hbm.at[pl.ds(base, EB)], etv, sem_a)
        d3 = pltpu.async_copy(attr_hbm.at[pl.ds(base * 2, EB * 2)], attrv,
                              sem_a)
        d0.wait()
        d1.wait()
        d2.wait()
        d3.wait()
        for g in range(EB // 16):
            sv = srcv[pl.ds(g * 16, 16)]
            dv = dstv[pl.ds(g * 16, 16)]
            tv = etv[pl.ds(g * 16, 16)]
            gsrcv[pl.ds(g * 16, 16)] = sv * 2 + c
            gdstv[pl.ds(g * 16, 16)] = dv * 2 + c
            dvv[pl.ds(g * 16, 16)] = jnp.where(tv == etype, dv, NN)
        ga = pltpu.async_copy(xl_hbm.at[gsrcv], xlb, sem_b)
        gb = pltpu.async_copy(xr_hbm.at[gdstv], xrb, sem_b)
        ga.wait()
        gb.wait()

        w0h = (wesv[pl.ds(0, 16)], wesv[pl.ds(16, 16)])
        w1h = (wesv[pl.ds(32, 16)], wesv[pl.ds(48, 16)])
        ath = (wesv[pl.ds(64, 16)], wesv[pl.ds(80, 16)])

        def grp(g, carry2):
            ap0 = attrv[pl.ds(g * 32, 16)]
            ap1 = attrv[pl.ds(g * 32 + 16, 16)]
            for j in range(16):
                e = g * 16 + j
                ap = ap0 if j < 8 else ap1
                a0 = ap[(2 * j) % 16]
                a1 = ap[(2 * j + 1) % 16]
                for h in range(2):
                    m = (xlb[e, pl.ds(16 * h, 16)] + xrb[e, pl.ds(16 * h, 16)]
                         + a0 * w0h[h] + a1 * w1h[h])
                    m = jnp.where(m >= 0.0, m, 0.2 * m)
                    cum = plsc.cumsum(m * ath[h])
                    plsc.store_scatter(
                        lgv, [jnp.full((16,), 16 * h + j, jnp.int32)], cum,
                        mask=lane15)
            ex0 = jnp.exp(lgv[pl.ds(0, 16)])
            ex1 = jnp.exp(lgv[pl.ds(16, 16)])
            row = g * 16 + iot
            plsc.store_scatter(dens, [row, zeros16], ex0)
            plsc.store_scatter(dens, [row, ones16], ex1)
            exh = (ex0, ex1)
            for j in range(16):
                e = g * 16 + j
                for h in range(2):
                    nums[e, pl.ds(16 * h, 16)] = (
                        exh[h][j] * xlb[e, pl.ds(16 * h, 16)])
            return 0

        lax.fori_loop(0, EB // 16, grp, 0)
        pltpu.sync_copy(nums, accn.at[dvv], add=True)
        pltpu.sync_copy(dens, accd.at[dvv], add=True)
        return 0

    lax.fori_loop(0, nblk, blk, 0)
    plsc.subcore_barrier()
    def out_blk(k, _):
        o = r0 + k * EB
        fill_dvv(o)
        pltpu.async_copy(accn.at[dvv], nums, sem_b).wait()
        pltpu.sync_copy(nums, num_out.at[c, pl.ds(o, EB)])
        pltpu.async_copy(accd.at[dvv], dens, sem_b).wait()
        pltpu.sync_copy(dens, den_out.at[c, pl.ds(o, EB)])
        return 0

    lax.fori_loop(0, rpt // EB, out_blk, 0)


@functools.cache
def _make_sc(etype, epad):
    ept = epad // NT
    nblk = ept // EB
    mesh = plsc.VectorSubcoreMesh(core_axis_name="c", subcore_axis_name="s")
    return pl.kernel(
        functools.partial(_sc_body, etype, ept, nblk),
        out_type=[jax.ShapeDtypeStruct((2, NP, 32), jnp.float32),
                  jax.ShapeDtypeStruct((2, NP, 2), jnp.float32)],
        mesh=mesh,
        compiler_params=pltpu.CompilerParams(
            needs_layout_passes=False, use_tc_tiling_on_sc=False),
        scratch_types=[
            pltpu.VMEM((EB,), jnp.int32),
            pltpu.VMEM((EB,), jnp.int32),
            pltpu.VMEM((EB,), jnp.int32),
            pltpu.VMEM((EB,), jnp.int32),
            pltpu.VMEM((EB,), jnp.int32),
            pltpu.VMEM((EB,), jnp.int32),
            pltpu.VMEM((EB * 2,), jnp.float32),
            pltpu.VMEM((EB, 32), jnp.float32),
            pltpu.VMEM((EB, 32), jnp.float32),
            pltpu.VMEM((EB, 32), jnp.float32),
            pltpu.VMEM((EB, 2), jnp.float32),
            pltpu.VMEM((96,), jnp.float32),
            pltpu.VMEM((32,), jnp.float32),
            pltpu.VMEM_SHARED((NP, 32), jnp.float32),
            pltpu.VMEM_SHARED((NP, 2), jnp.float32),
            pltpu.SemaphoreType.DMA,
            pltpu.SemaphoreType.DMA,
        ],
        name="gat_edge_pass_t%d" % etype,
    )


def _mean_body(a0_ref, a1_ref, et_ref, out_ref):
    i = pl.program_id(0)

    @pl.when(i == 0)
    def _():
        out_ref[...] = jnp.zeros_like(out_ref)

    a0 = a0_ref[0]
    a1 = a1_ref[0]
    et = et_ref[0]
    m0 = (et == 0).astype(jnp.float32)
    m1 = (et == 1).astype(jnp.float32)
    lane = lax.broadcasted_iota(jnp.int32, (1, 128), 1)
    z = jnp.zeros((1, 128), jnp.float32)
    vals = (jnp.where(lane == 0, jnp.sum(a0 * m0), z)
            + jnp.where(lane == 1, jnp.sum(a1 * m0), z)
            + jnp.where(lane == 2, jnp.sum(m0), z)
            + jnp.where(lane == 3, jnp.sum(a0 * m1), z)
            + jnp.where(lane == 4, jnp.sum(a1 * m1), z)
            + jnp.where(lane == 5, jnp.sum(m1), z))
    out_ref[...] += vals


def _edge_means(edge_type, edge_attr):
    e = edge_type.shape[0]
    be = 8000
    g = e // be
    a0 = edge_attr[:, 0].reshape(g, 1, be)
    a1 = edge_attr[:, 1].reshape(g, 1, be)
    et = edge_type.reshape(g, 1, be)
    sums = pl.pallas_call(
        _mean_body,
        grid=(g,),
        in_specs=[pl.BlockSpec((1, 1, be), lambda i: (i, 0, 0))] * 3,
        out_specs=pl.BlockSpec((1, 128), lambda i: (0, 0)),
        out_shape=jax.ShapeDtypeStruct((1, 128), jnp.float32),
    )(a0, a1, et)
    ma0 = sums[:, 0:2] / sums[0, 2]
    ma1 = sums[:, 3:5] / sums[0, 5]
    return ma0, ma1


def _prep_body(x_ref, fi_ref, ci_ref, fe_ref, ce_ref, ew_ref, eb_ref, pw_ref,
               w0l_ref, w0r_ref, w1l_ref, w1r_ref,
               xp_ref, xl0_ref, xr0_ref, xl1_ref, xr1_ref):
    b = x_ref.shape[0]
    fi = fi_ref[0]
    ci = ci_ref[0]
    ohf = (lax.broadcasted_iota(jnp.int32, (128, b), 0) == fi
           ).astype(jnp.float32)
    ohc = (lax.broadcasted_iota(jnp.int32, (32, b), 0) == ci
           ).astype(jnp.float32)
    dn = (((0,), (0,)), ((), ()))
    fe = lax.dot_general(ohf, fe_ref[...], dn,
                         preferred_element_type=jnp.float32)
    ce = lax.dot_general(ohc, ce_ref[...], dn,
                         preferred_element_type=jnp.float32)
    ew = ew_ref[...]
    xf = (x_ref[...] @ ew[:CD] + fe @ ew[CD:CD + 8] + ce @ ew[CD + 8:CD + 16]
          + eb_ref[...])
    xp_ref[...] = xf @ pw_ref[...]
    xl0_ref[...] = xf @ w0l_ref[...]
    xr0_ref[...] = xf @ w0r_ref[...]
    xl1_ref[...] = xf @ w1l_ref[...]
    xr1_ref[...] = xf @ w1r_ref[...]


def _prep(x, flag_idx, class_idx, p):
    nb = NN // BN
    cep = jnp.zeros((32, 8), jnp.float32).at[:17].set(p['class_emb'])
    fi3 = flag_idx.reshape(nb, 1, BN)
    ci3 = class_idx.reshape(nb, 1, BN)
    full = lambda shp: pl.BlockSpec(shp, lambda i: tuple(0 for _ in shp))
    row = lambda d: pl.BlockSpec((BN, d), lambda i: (i, 0))
    outs = pl.pallas_call(
        _prep_body,
        grid=(nb,),
        in_specs=[row(CD),
                  pl.BlockSpec((1, 1, BN), lambda i: (i, 0, 0)),
                  pl.BlockSpec((1, 1, BN), lambda i: (i, 0, 0)),
                  full((128, 8)), full((32, 8)), full((56, CD)),
                  full((1, CD)), full((CD, HID)),
                  full((CD, HID)), full((CD, HID)),
                  full((CD, HID)), full((CD, HID))],
        out_specs=[row(HID)] * 5,
        out_shape=[jax.ShapeDtypeStruct((NN, HID), jnp.float32)] * 5,
    )(x, fi3, ci3, p['flag_emb'], cep, p['emb_W'],
      p['emb_b'].reshape(1, CD), p['input_proj_W'],
      p['conv1_0']['Wl'], p['conv1_0']['Wr'],
      p['conv1_1']['Wl'], p['conv1_1']['Wr'])
    return outs


def _post_body(mid, num0_ref, num1_ref, den0_ref, den1_ref,
               xl_ref, xr_ref, res_ref,
               ma_ref, we_ref, att_ref, bias_ref, g_ref, b_ref, *rest):
    num = jnp.concatenate([num0_ref[...], num1_ref[...]], axis=1)
    den = jnp.concatenate([den0_ref[...], den1_ref[...]], axis=1)
    xl = xl_ref[...]
    xr = xr_ref[...]
    es = lax.dot_general(ma_ref[...], we_ref[...], (((1,), (0,)), ((), ())),
                         preferred_element_type=jnp.float32)
    ms = xl + xr + es
    ms = jnp.where(ms >= 0.0, ms, 0.2 * ms)
    sel = (lax.broadcasted_iota(jnp.int32, (HID, HEADS), 0) // CH
           == lax.broadcasted_iota(jnp.int32, (HID, HEADS), 1)
           ).astype(jnp.float32)
    dn_c1 = (((1,), (0,)), ((), ()))
    dn_c11 = (((1,), (1,)), ((), ()))
    ls = lax.dot_general(ms * att_ref[...], sel, dn_c1,
                         preferred_element_type=jnp.float32)
    exs = jnp.exp(ls)
    den4 = den + exs
    dexp = lax.dot_general(exs, sel, dn_c11,
                           preferred_element_type=jnp.float32)
    denx = lax.dot_general(den4, sel, dn_c11,
                           preferred_element_type=jnp.float32)
    out = (num + dexp * xl) / denx + bias_ref[...]
    hv = jnp.where(out > 0.0, out, jnp.exp(out) - 1.0)
    t = hv + res_ref[...]
    mu = jnp.mean(t, axis=-1, keepdims=True)
    var = jnp.mean((t - mu) ** 2, axis=-1, keepdims=True)
    hn = (t - mu) / jnp.sqrt(var + 1e-5) * g_ref[...] + b_ref[...]
    if mid:
        w2l_ref, w2r_ref, h_ref, xl2_ref, xr2_ref = rest
        h_ref[...] = hn
        xl2_ref[...] = hn @ w2l_ref[...]
        xr2_ref[...] = hn @ w2r_ref[...]
    else:
        rest[0][...] = hn


def _post(mid, num, den, xlt, xrt, res, ma, cp, gamma, beta, w2=None):
    nb = NN // BN
    full = lambda shp: pl.BlockSpec(shp, lambda i: tuple(0 for _ in shp))
    row = lambda d: pl.BlockSpec((BN, d), lambda i: (i, 0))
    n_out = 3 if mid else 1
    ins = [num[0], num[1], den[0], den[1], xlt, xrt, res, ma, cp['We'],
           cp['att'].reshape(1, HID), cp['bias'].reshape(1, HID),
           gamma.reshape(1, HID), beta.reshape(1, HID)]
    in_specs = [row(32), row(32), row(2), row(2), row(HID), row(HID),
                row(HID),
                full((1, 2)), full((2, HID)), full((1, HID)), full((1, HID)),
                full((1, HID)), full((1, HID))]
    if mid:
        ins += [w2['Wl'], w2['Wr']]
        in_specs += [full((HID, HID)), full((HID, HID))]
    outs = pl.pallas_call(
        functools.partial(_post_body, mid),
        grid=(nb,),
        in_specs=in_specs,
        out_specs=[row(HID)] * n_out,
        out_shape=[jax.ShapeDtypeStruct((NN, HID), jnp.float32)] * n_out,
    )(*ins)
    return outs if mid else outs[0]


def _final_body(h0_ref, h1_ref, tw_ref, pw_ref, pb_ref,
                wr_ref, wz_ref, wn_ref, br_ref, bz_ref, bn_ref,
                bhr_ref, bhz_ref, bhn_ref, gg_ref, gb_ref,
                hw_ref, hb_ref, o_ref):
    o64 = h0_ref[...] * tw_ref[0, 0] + h1_ref[...] * tw_ref[0, 1]
    s = o64 @ pw_ref[...] + pb_ref[...]
    r = jax.nn.sigmoid(s @ wr_ref[...] + br_ref[...] + bhr_ref[...])
    z = jax.nn.sigmoid(s @ wz_ref[...] + bz_ref[...] + bhz_ref[...])
    nc = jnp.tanh(s @ wn_ref[...] + bn_ref[...] + r * bhn_ref[...])
    hn = (1.0 - z) * nc
    mu = jnp.mean(hn, axis=-1, keepdims=True)
    var = jnp.mean((hn - mu) ** 2, axis=-1, keepdims=True)
    hn = (hn - mu) / jnp.sqrt(var + 1e-5) * gg_ref[...] + gb_ref[...]
    o_ref[...] = hn @ hw_ref[...] + hb_ref[...]


def _final(h20, h21, p):
    nb = NN // BN
    tw = jax.nn.softmax(p['type_weights']).reshape(1, 2)
    full = lambda shp: pl.BlockSpec(shp, lambda i: tuple(0 for _ in shp))
    row = lambda d: pl.BlockSpec((BN, d), lambda i: (i, 0))
    wih = p['gru_Wih']
    bih = p['gru_bih'].reshape(1, 3 * HID)
    bhh = p['gru_bhh'].reshape(1, 3 * HID)
    return pl.pallas_call(
        _final_body,
        grid=(nb,),
        in_specs=[row(HID), row(HID),
                  pl.BlockSpec(memory_space=pltpu.SMEM),
                  full((HID, HID)), full((1, HID)),
                  full((HID, HID)), full((HID, HID)), full((HID, HID)),
                  full((1, HID)), full((1, HID)), full((1, HID)),
                  full((1, HID)), full((1, HID)), full((1, HID)),
                  full((1, HID)), full((1, HID)),
                  full((HID, 4)), full((1, 4))],
        out_specs=row(4),
        out_shape=jax.ShapeDtypeStruct((NN, 4), jnp.float32),
    )(h20, h21, tw, p['proj_W'], p['proj_b'].reshape(1, HID),
      wih[:, :HID], wih[:, HID:2 * HID], wih[:, 2 * HID:],
      bih[:, :HID], bih[:, HID:2 * HID], bih[:, 2 * HID:],
      bhh[:, :HID], bhh[:, HID:2 * HID], bhh[:, 2 * HID:],
      p['gru_norm_g'].reshape(1, HID), p['gru_norm_b'].reshape(1, HID),
      p['head_W'], p['head_b'].reshape(1, 4))


def _pack_par(cp):
    we = cp['We']
    attf = cp['att'].reshape(HID)
    return jnp.stack([
        jnp.concatenate([we[0, :32], we[1, :32], attf[:32]]),
        jnp.concatenate([we[0, 32:], we[1, 32:], attf[32:]]),
    ])


def kernel(x, edge_index, edge_type, edge_attr, flag_idx, class_idx, params):
    e = edge_index.shape[1]
    quant = NT * EB
    epad = ((e + quant - 1) // quant) * quant
    pad = epad - e
    if pad:
        zpad = jnp.zeros((pad,), jnp.int32)
        src_p = jnp.concatenate([edge_index[0], zpad])
        dst_p = jnp.concatenate([edge_index[1], zpad])
        et_p = jnp.concatenate([edge_type, zpad - 1])
        attr_p = jnp.concatenate(
            [edge_attr, jnp.zeros((pad, 2), jnp.float32)])
    else:
        src_p, dst_p, et_p, attr_p = (
            edge_index[0], edge_index[1], edge_type, edge_attr)
    attr_flat = attr_p.reshape(-1)

    ma = _edge_means(edge_type, edge_attr)
    xp, xl0, xr0, xl1, xr1 = _prep(x, flag_idx, class_idx, params)

    h2n = []
    for et, xlt, xrt in ((0, xl0, xr0), (1, xl1, xr1)):
        sck = _make_sc(et, epad)
        res = xp
        for li in range(1, 3):
            cp = params['conv%d_%d' % (li, et)]
            par = _pack_par(cp)
            num, den = sck(src_p, dst_p, et_p, attr_flat,
                           xlt.reshape(2 * NN, 32), xrt.reshape(2 * NN, 32),
                           par)
            g = params['norm%d_%d_g' % (li, et)]
            b = params['norm%d_%d_b' % (li, et)]
            if li == 1:
                h, xlt, xrt = _post(True, num, den, xlt, xrt, res,
                                    ma[et], cp, g, b,
                                    params['conv2_%d' % et])
                res = h
            else:
                h2n.append(_post(False, num, den, xlt, xrt, res,
                                 ma[et], cp, g, b))
    return _final(h2n[0], h2n[1], params)
